# baseline (speedup 1.0000x reference)
import functools

import jax
import jax.numpy as jnp
from jax import lax
from jax.experimental import pallas as pl

N_REL = 100000
FEAT = 353
NCLS = 151
EDGE = 51
LBLK = 2048
GRID = (N_REL + LBLK - 1) // LBLK
BIGF = float(2 ** 24)

_HI = lax.Precision.HIGHEST


def _dotT(a, b):
    return lax.dot_general(a, b, (((1,), (1,)), ((), ())),
                           precision=_HI, preferred_element_type=jnp.float32)


def _dotL(a, b):
    return lax.dot_general(a, b, (((1,), (1,)), ((), ())),
                           preferred_element_type=jnp.float32)


def _dotC0(a, b):
    return lax.dot_general(a, b, (((0,), (0,)), ((), ())),
                           precision=_HI, preferred_element_type=jnp.float32)


def _stream_kernel(x_ref, c_ref, e_ref, ft_ref, x0_ref):
    pid = pl.program_id(0)

    @pl.when(pid == 0)
    def _init():
        c_ref[...] = jnp.zeros_like(c_ref)
        e_ref[...] = jnp.zeros_like(e_ref)
        ft_ref[...] = jnp.full_like(ft_ref, BIGF)
        x0_ref[...] = jnp.zeros_like(x0_ref)

    lane = lax.broadcasted_iota(jnp.int32, (1, LBLK), 1)
    nvalid = jnp.minimum(N_REL - pid * LBLK, LBLK)
    valid = lane < nvalid
    x = jnp.where(valid, x_ref[...], 0.0)
    rowc = lax.broadcasted_iota(jnp.int32, (FEAT, 1), 0)
    NEG = -3.0e38
    mask_s = jnp.where((rowc >= 51) & (rowc < 202), 0.0, NEG)
    mask_o = jnp.where(rowc >= 202, 0.0, NEG)
    ps_m = x + mask_s
    po_m = x + mask_o
    ps_max = jnp.max(ps_m, axis=0, keepdims=True)
    po_max = jnp.max(po_m, axis=0, keepdims=True)
    eq_s = (ps_m == ps_max) & valid
    eq_o = (po_m == po_max) & valid
    s_bf = eq_s.astype(jnp.bfloat16)
    d_bf = eq_o.astype(jnp.bfloat16)

    C_blk = _dotL(d_bf, s_bf)
    ea = x[0:EDGE, :]
    ea_hi = ea.astype(jnp.bfloat16)
    ea_lo = (ea - ea_hi.astype(jnp.float32)).astype(jnp.bfloat16)
    E_blk = _dotL(d_bf, ea_hi) + _dotL(d_bf, ea_lo)

    c_ref[...] += C_blk
    e_ref[...] += E_blk

    BIGI = 2 ** 24

    total = jnp.sum(C_blk)

    @pl.when(total != nvalid.astype(jnp.float32))
    def _tie_fixup():
        s_row = jnp.min(jnp.where(eq_s, rowc, BIGI), axis=0, keepdims=True)
        o_row = jnp.min(jnp.where(eq_o, rowc, BIGI), axis=0, keepdims=True)
        s_ex = ((rowc == s_row) & valid).astype(jnp.bfloat16)
        d_ex = ((rowc == o_row) & valid).astype(jnp.bfloat16)
        C_ex = _dotL(d_ex, s_ex)
        E_ex = _dotL(d_ex, ea_hi) + _dotL(d_ex, ea_lo)
        c_ref[...] += C_ex - C_blk
        e_ref[...] += E_ex - E_blk

    prev = ft_ref[...]

    @pl.when(jnp.max(prev) >= BIGF)
    def _first_occurrence():
        s_row = jnp.min(jnp.where(eq_s, rowc, BIGI), axis=0, keepdims=True)
        o_row = jnp.min(jnp.where(eq_o, rowc, BIGI), axis=0, keepdims=True)
        s_ex = (rowc == s_row) & valid
        d_ex = (rowc == o_row) & valid
        ex = s_ex | d_ex
        coli = lane + pid * LBLK
        cmin = jnp.min(jnp.where(ex, coli, BIGI), axis=1, keepdims=True)
        par = jnp.where(rowc >= 202, 1, 0)
        cand = jnp.where(cmin < BIGI, (2 * cmin + par).astype(jnp.float32), BIGF)
        newly = cand < prev
        G = (ex & (coli == cmin) & newly)
        g_bf = G.astype(jnp.bfloat16)
        x_hi = x.astype(jnp.bfloat16)
        x_lo = (x - x_hi.astype(jnp.float32)).astype(jnp.bfloat16)
        x0_new = _dotL(g_bf, x_hi) + _dotL(g_bf, x_lo)
        m = newly.astype(jnp.float32)
        x0_ref[...] = x0_ref[...] * (1.0 - m) + x0_new * m
        ft_ref[...] = jnp.minimum(prev, cand)


def _finish_kernel(c_ref, e_ref, ft_ref, x0_ref, lr_ref,
                   wea1_ref, bea1_ref, wl1_ref, bl1_ref,
                   wea2_ref, bea2_ref, wl2_ref, bl2_ref,
                   wout_ref, bout_ref, out_ref):
    j353 = lax.broadcasted_iota(jnp.int32, (FEAT, NCLS), 0)
    c353 = lax.broadcasted_iota(jnp.int32, (FEAT, NCLS), 1)
    Esub = (j353 == c353 + 51).astype(jnp.float32)
    Eobj = (j353 == c353 + 202).astype(jnp.float32)

    def mm(a, b):
        return lax.dot_general(a, b, (((1,), (0,)), ((), ())),
                               precision=_HI, preferred_element_type=jnp.float32)

    ftw = ft_ref[...]
    ft_s = _dotC0(Esub, ftw)
    ft_o = _dotC0(Eobj, ftw)
    ftT = jnp.minimum(ft_s, ft_o)
    is_sub = (ft_s <= ft_o).astype(jnp.float32)
    seen = (ftT < BIGF).astype(jnp.float32)
    ft = jnp.transpose(ftT, (1, 0))

    cls_r = lax.broadcasted_iota(jnp.int32, (NCLS, NCLS), 1)
    cls_c = lax.broadcasted_iota(jnp.int32, (NCLS, NCLS), 0)
    cmp = (ftT < ft) | ((ftT == ft) & (cls_c < cls_r))
    rank = jnp.sum(cmp.astype(jnp.int32), axis=0, keepdims=True)
    P = (lax.broadcasted_iota(jnp.int32, (NCLS, NCLS), 0) == rank)
    Pf = P.astype(jnp.float32)

    Cw = c_ref[...]
    C = mm(_dotC0(Eobj, Cw), Esub)
    E = _dotC0(Eobj, e_ref[...])

    x0w = x0_ref[...]
    x0_s = mm(_dotC0(Esub, x0w), Esub)
    x0_o = mm(_dotC0(Eobj, x0w), Eobj)
    x0 = x0_s * is_sub + x0_o * (1.0 - is_sub)
    x0 = x0 * seen + mm(lr_ref[...], Esub) * (1.0 - seen)

    cnt1 = jnp.sum(C, axis=1, keepdims=True) + 1.0

    agg1 = mm(C, x0) + x0 + _dotT(E, wea1_ref[...]) + cnt1 * bea1_ref[...]
    x1 = _dotT(agg1, wl1_ref[...]) + bl1_ref[...]
    agg2 = mm(C, x1) + x1 + _dotT(E, wea2_ref[...]) + cnt1 * bea2_ref[...]
    x2 = _dotT(agg2, wl2_ref[...]) + bl2_ref[...]
    oc = _dotT(x2, wout_ref[...]) + bout_ref[...]
    out_ref[...] = mm(Pf, oc)


@functools.partial(jax.jit, static_argnames=("interpret",))
def _run(scene_feat, W_ea1, b_ea1, W_lin1, b_lin1, W_ea2, b_ea2,
         W_lin2, b_lin2, W_out, b_out, interpret=False):
    f32 = jnp.float32
    sft = scene_feat.T
    Cw, Ew, ftw, x0w = pl.pallas_call(
        _stream_kernel,
        grid=(GRID,),
        in_specs=[pl.BlockSpec((FEAT, LBLK), lambda i: (0, i))],
        out_specs=[
            pl.BlockSpec((FEAT, FEAT), lambda i: (0, 0)),
            pl.BlockSpec((FEAT, EDGE), lambda i: (0, 0)),
            pl.BlockSpec((FEAT, 1), lambda i: (0, 0)),
            pl.BlockSpec((FEAT, FEAT), lambda i: (0, 0)),
        ],
        out_shape=[
            jax.ShapeDtypeStruct((FEAT, FEAT), f32),
            jax.ShapeDtypeStruct((FEAT, EDGE), f32),
            jax.ShapeDtypeStruct((FEAT, 1), f32),
            jax.ShapeDtypeStruct((FEAT, FEAT), f32),
        ],
        interpret=interpret,
    )(sft)

    last_row = lax.slice(scene_feat, (N_REL - 1, 0), (N_REL, FEAT))
    out = pl.pallas_call(
        _finish_kernel,
        out_shape=jax.ShapeDtypeStruct((NCLS, NCLS), f32),
        interpret=interpret,
    )(Cw, Ew, ftw, x0w, last_row,
      W_ea1, b_ea1.reshape(1, -1), W_lin1, b_lin1.reshape(1, -1),
      W_ea2, b_ea2.reshape(1, -1), W_lin2, b_lin2.reshape(1, -1),
      W_out, b_out.reshape(1, -1))
    return out


def kernel(scene_feat, W_ea1, b_ea1, W_lin1, b_lin1, W_ea2, b_ea2,
           W_lin2, b_lin2, W_out, b_out):
    return _run(scene_feat, W_ea1, b_ea1, W_lin1, b_lin1,
                W_ea2, b_ea2, W_lin2, b_lin2, W_out, b_out)


# speedup vs baseline: 2.2586x; 2.2586x over previous
import functools

import jax
import jax.numpy as jnp
from jax import lax
from jax.experimental import pallas as pl

N_REL = 100000
FEAT = 353
NCLS = 151
EDGE = 51
LBLK = 2048
GRID = (N_REL + LBLK - 1) // LBLK
BIGF = float(2 ** 24)

_HI = lax.Precision.HIGHEST


def _dotT(a, b):
    return lax.dot_general(a, b, (((1,), (1,)), ((), ())),
                           precision=_HI, preferred_element_type=jnp.float32)


def _dotL(a, b):
    return lax.dot_general(a, b, (((1,), (1,)), ((), ())),
                           preferred_element_type=jnp.float32)


def _dotC0(a, b):
    return lax.dot_general(a, b, (((0,), (0,)), ((), ())),
                           precision=_HI, preferred_element_type=jnp.float32)


def _stream_kernel(x_ref, c_ref, e_ref, ft_ref, x0_ref):
    pid = pl.program_id(0)

    @pl.when(pid == 0)
    def _init():
        c_ref[...] = jnp.zeros_like(c_ref)
        e_ref[...] = jnp.zeros_like(e_ref)
        r = lax.broadcasted_iota(jnp.int32, (FEAT, 1), 0)
        ft_ref[...] = jnp.where(r >= 51, BIGF, 0.0)
        x0_ref[...] = jnp.zeros_like(x0_ref)

    lane = lax.broadcasted_iota(jnp.int32, (1, LBLK), 1)
    nvalid = jnp.minimum(N_REL - pid * LBLK, LBLK)
    valid = lane < nvalid
    x = jnp.where(valid, x_ref[...], 0.0)
    rowc = lax.broadcasted_iota(jnp.int32, (FEAT, 1), 0)
    NEG = -3.0e38
    mask_s = jnp.where((rowc >= 51) & (rowc < 202), 0.0, NEG)
    mask_o = jnp.where(rowc >= 202, 0.0, NEG)
    ps_m = x + mask_s
    po_m = x + mask_o
    ps_max = jnp.max(ps_m, axis=0, keepdims=True)
    po_max = jnp.max(po_m, axis=0, keepdims=True)
    eq_s = (ps_m == ps_max) & valid
    eq_o = (po_m == po_max) & valid
    s_bf = eq_s.astype(jnp.bfloat16)
    d_bf = eq_o.astype(jnp.bfloat16)

    C_blk = _dotL(d_bf, s_bf)
    ea = x[0:EDGE, :]
    ea_hi = ea.astype(jnp.bfloat16)
    ea_lo = (ea - ea_hi.astype(jnp.float32)).astype(jnp.bfloat16)
    E_blk = _dotL(d_bf, ea_hi) + _dotL(d_bf, ea_lo)

    c_ref[...] += C_blk
    e_ref[...] += E_blk

    BIGI = 2 ** 24

    total = jnp.sum(C_blk)

    @pl.when(total != nvalid.astype(jnp.float32))
    def _tie_fixup():
        s_row = jnp.min(jnp.where(eq_s, rowc, BIGI), axis=0, keepdims=True)
        o_row = jnp.min(jnp.where(eq_o, rowc, BIGI), axis=0, keepdims=True)
        s_ex = ((rowc == s_row) & valid).astype(jnp.bfloat16)
        d_ex = ((rowc == o_row) & valid).astype(jnp.bfloat16)
        C_ex = _dotL(d_ex, s_ex)
        E_ex = _dotL(d_ex, ea_hi) + _dotL(d_ex, ea_lo)
        c_ref[...] += C_ex - C_blk
        e_ref[...] += E_ex - E_blk

    prev = ft_ref[...]

    @pl.when(jnp.max(prev) >= BIGF)
    def _first_occurrence():
        s_row = jnp.min(jnp.where(eq_s, rowc, BIGI), axis=0, keepdims=True)
        o_row = jnp.min(jnp.where(eq_o, rowc, BIGI), axis=0, keepdims=True)
        s_ex = (rowc == s_row) & valid
        d_ex = (rowc == o_row) & valid
        ex = s_ex | d_ex
        coli = lane + pid * LBLK
        cmin = jnp.min(jnp.where(ex, coli, BIGI), axis=1, keepdims=True)
        par = jnp.where(rowc >= 202, 1, 0)
        cand = jnp.where(cmin < BIGI, (2 * cmin + par).astype(jnp.float32), BIGF)
        newly = cand < prev
        G = (ex & (coli == cmin) & newly)
        g_bf = G.astype(jnp.bfloat16)
        x_hi = x.astype(jnp.bfloat16)
        x_lo = (x - x_hi.astype(jnp.float32)).astype(jnp.bfloat16)
        x0_new = _dotL(g_bf, x_hi) + _dotL(g_bf, x_lo)
        m = newly.astype(jnp.float32)
        x0_ref[...] = x0_ref[...] * (1.0 - m) + x0_new * m
        ft_ref[...] = jnp.minimum(prev, cand)


def _finish_kernel(c_ref, e_ref, ft_ref, x0_ref, lr_ref,
                   wea1_ref, bea1_ref, wl1_ref, bl1_ref,
                   wea2_ref, bea2_ref, wl2_ref, bl2_ref,
                   wout_ref, bout_ref, out_ref):
    j353 = lax.broadcasted_iota(jnp.int32, (FEAT, NCLS), 0)
    c353 = lax.broadcasted_iota(jnp.int32, (FEAT, NCLS), 1)
    Esub = (j353 == c353 + 51).astype(jnp.float32)
    Eobj = (j353 == c353 + 202).astype(jnp.float32)

    def mm(a, b):
        return lax.dot_general(a, b, (((1,), (0,)), ((), ())),
                               precision=_HI, preferred_element_type=jnp.float32)

    ftw = ft_ref[...]
    ft_s = _dotC0(Esub, ftw)
    ft_o = _dotC0(Eobj, ftw)
    ftT = jnp.minimum(ft_s, ft_o)
    is_sub = (ft_s <= ft_o).astype(jnp.float32)
    seen = (ftT < BIGF).astype(jnp.float32)
    ft = jnp.transpose(ftT, (1, 0))

    cls_r = lax.broadcasted_iota(jnp.int32, (NCLS, NCLS), 1)
    cls_c = lax.broadcasted_iota(jnp.int32, (NCLS, NCLS), 0)
    cmp = (ftT < ft) | ((ftT == ft) & (cls_c < cls_r))
    rank = jnp.sum(cmp.astype(jnp.int32), axis=0, keepdims=True)
    P = (lax.broadcasted_iota(jnp.int32, (NCLS, NCLS), 0) == rank)
    Pf = P.astype(jnp.float32)

    Cw = c_ref[...]
    C = mm(_dotC0(Eobj, Cw), Esub)
    E = _dotC0(Eobj, e_ref[...])

    x0w = x0_ref[...]
    x0_s = mm(_dotC0(Esub, x0w), Esub)
    x0_o = mm(_dotC0(Eobj, x0w), Eobj)
    x0 = x0_s * is_sub + x0_o * (1.0 - is_sub)
    x0 = x0 * seen + mm(lr_ref[...], Esub) * (1.0 - seen)

    cnt1 = jnp.sum(C, axis=1, keepdims=True) + 1.0

    agg1 = mm(C, x0) + x0 + _dotT(E, wea1_ref[...]) + cnt1 * bea1_ref[...]
    x1 = _dotT(agg1, wl1_ref[...]) + bl1_ref[...]
    agg2 = mm(C, x1) + x1 + _dotT(E, wea2_ref[...]) + cnt1 * bea2_ref[...]
    x2 = _dotT(agg2, wl2_ref[...]) + bl2_ref[...]
    oc = _dotT(x2, wout_ref[...]) + bout_ref[...]
    out_ref[...] = mm(Pf, oc)


@functools.partial(jax.jit, static_argnames=("interpret",))
def _run(scene_feat, W_ea1, b_ea1, W_lin1, b_lin1, W_ea2, b_ea2,
         W_lin2, b_lin2, W_out, b_out, interpret=False):
    f32 = jnp.float32
    sft = scene_feat.T
    Cw, Ew, ftw, x0w = pl.pallas_call(
        _stream_kernel,
        grid=(GRID,),
        in_specs=[pl.BlockSpec((FEAT, LBLK), lambda i: (0, i))],
        out_specs=[
            pl.BlockSpec((FEAT, FEAT), lambda i: (0, 0)),
            pl.BlockSpec((FEAT, EDGE), lambda i: (0, 0)),
            pl.BlockSpec((FEAT, 1), lambda i: (0, 0)),
            pl.BlockSpec((FEAT, FEAT), lambda i: (0, 0)),
        ],
        out_shape=[
            jax.ShapeDtypeStruct((FEAT, FEAT), f32),
            jax.ShapeDtypeStruct((FEAT, EDGE), f32),
            jax.ShapeDtypeStruct((FEAT, 1), f32),
            jax.ShapeDtypeStruct((FEAT, FEAT), f32),
        ],
        interpret=interpret,
    )(sft)

    last_row = lax.slice(scene_feat, (N_REL - 1, 0), (N_REL, FEAT))
    out = pl.pallas_call(
        _finish_kernel,
        out_shape=jax.ShapeDtypeStruct((NCLS, NCLS), f32),
        interpret=interpret,
    )(Cw, Ew, ftw, x0w, last_row,
      W_ea1, b_ea1.reshape(1, -1), W_lin1, b_lin1.reshape(1, -1),
      W_ea2, b_ea2.reshape(1, -1), W_lin2, b_lin2.reshape(1, -1),
      W_out, b_out.reshape(1, -1))
    return out


def kernel(scene_feat, W_ea1, b_ea1, W_lin1, b_lin1, W_ea2, b_ea2,
           W_lin2, b_lin2, W_out, b_out):
    return _run(scene_feat, W_ea1, b_ea1, W_lin1, b_lin1,
                W_ea2, b_ea2, W_lin2, b_lin2, W_out, b_out)


# speedup vs baseline: 2.3824x; 1.0548x over previous
import functools

import jax
import jax.numpy as jnp
from jax import lax
from jax.experimental import pallas as pl

N_REL = 100000
FEAT = 353
NCLS = 151
EDGE = 51
LBLK = 4096
GRID = (N_REL + LBLK - 1) // LBLK
BIGF = float(2 ** 24)

_HI = lax.Precision.HIGHEST


def _dotT(a, b):
    return lax.dot_general(a, b, (((1,), (1,)), ((), ())),
                           precision=_HI, preferred_element_type=jnp.float32)


def _dotL(a, b):
    return lax.dot_general(a, b, (((1,), (1,)), ((), ())),
                           preferred_element_type=jnp.float32)


def _dotC0(a, b):
    return lax.dot_general(a, b, (((0,), (0,)), ((), ())),
                           precision=_HI, preferred_element_type=jnp.float32)


def _stream_kernel(x_ref, c_ref, e_ref, ft_ref, x0_ref):
    pid = pl.program_id(0)

    @pl.when(pid == 0)
    def _init():
        c_ref[...] = jnp.zeros_like(c_ref)
        e_ref[...] = jnp.zeros_like(e_ref)
        r = lax.broadcasted_iota(jnp.int32, (FEAT, 1), 0)
        ft_ref[...] = jnp.where(r >= 51, BIGF, 0.0)
        x0_ref[...] = jnp.zeros_like(x0_ref)

    lane = lax.broadcasted_iota(jnp.int32, (1, LBLK), 1)
    nvalid = jnp.minimum(N_REL - pid * LBLK, LBLK)
    valid = lane < nvalid
    x = x_ref[...]
    rowc = lax.broadcasted_iota(jnp.int32, (FEAT, 1), 0)
    NEG = -3.0e38
    mask_s = jnp.where((rowc >= 51) & (rowc < 202), 0.0, NEG)
    mask_o = jnp.where(rowc >= 202, 0.0, NEG)
    ps_m = x + mask_s
    po_m = x + mask_o
    ps_max = jnp.max(ps_m, axis=0, keepdims=True)
    po_max = jnp.max(po_m, axis=0, keepdims=True)
    eq_s = (ps_m == ps_max) & valid
    eq_o = (po_m == po_max) & valid
    s_bf = eq_s.astype(jnp.bfloat16)
    d_bf = eq_o.astype(jnp.bfloat16)

    C_blk = _dotL(d_bf, s_bf)
    ea = jnp.where(valid, x[0:EDGE, :], 0.0)
    ea_hi = ea.astype(jnp.bfloat16)
    ea_lo = (ea - ea_hi.astype(jnp.float32)).astype(jnp.bfloat16)
    E_blk = _dotL(d_bf, ea_hi) + _dotL(d_bf, ea_lo)

    c_ref[...] += C_blk
    e_ref[...] += E_blk

    BIGI = 2 ** 24

    total = jnp.sum(C_blk)

    @pl.when(total != nvalid.astype(jnp.float32))
    def _tie_fixup():
        s_row = jnp.min(jnp.where(eq_s, rowc, BIGI), axis=0, keepdims=True)
        o_row = jnp.min(jnp.where(eq_o, rowc, BIGI), axis=0, keepdims=True)
        s_ex = ((rowc == s_row) & valid).astype(jnp.bfloat16)
        d_ex = ((rowc == o_row) & valid).astype(jnp.bfloat16)
        C_ex = _dotL(d_ex, s_ex)
        E_ex = _dotL(d_ex, ea_hi) + _dotL(d_ex, ea_lo)
        c_ref[...] += C_ex - C_blk
        e_ref[...] += E_ex - E_blk

    prev = ft_ref[...]

    @pl.when(jnp.max(prev) >= BIGF)
    def _first_occurrence():
        s_row = jnp.min(jnp.where(eq_s, rowc, BIGI), axis=0, keepdims=True)
        o_row = jnp.min(jnp.where(eq_o, rowc, BIGI), axis=0, keepdims=True)
        s_ex = (rowc == s_row) & valid
        d_ex = (rowc == o_row) & valid
        ex = s_ex | d_ex
        coli = lane + pid * LBLK
        cmin = jnp.min(jnp.where(ex, coli, BIGI), axis=1, keepdims=True)
        par = jnp.where(rowc >= 202, 1, 0)
        cand = jnp.where(cmin < BIGI, (2 * cmin + par).astype(jnp.float32), BIGF)
        newly = cand < prev
        G = (ex & (coli == cmin) & newly)
        g_bf = G.astype(jnp.bfloat16)
        xz = jnp.where(valid, x, 0.0)
        x_hi = xz.astype(jnp.bfloat16)
        x_lo = (xz - x_hi.astype(jnp.float32)).astype(jnp.bfloat16)
        x0_new = _dotL(g_bf, x_hi) + _dotL(g_bf, x_lo)
        m = newly.astype(jnp.float32)
        x0_ref[...] = x0_ref[...] * (1.0 - m) + x0_new * m
        ft_ref[...] = jnp.minimum(prev, cand)


def _finish_kernel(c_ref, e_ref, ft_ref, x0_ref, lr_ref,
                   wea1_ref, bea1_ref, wl1_ref, bl1_ref,
                   wea2_ref, bea2_ref, wl2_ref, bl2_ref,
                   wout_ref, bout_ref, out_ref):
    j353 = lax.broadcasted_iota(jnp.int32, (FEAT, NCLS), 0)
    c353 = lax.broadcasted_iota(jnp.int32, (FEAT, NCLS), 1)
    Esub = (j353 == c353 + 51).astype(jnp.float32)
    Eobj = (j353 == c353 + 202).astype(jnp.float32)

    def mm(a, b):
        return lax.dot_general(a, b, (((1,), (0,)), ((), ())),
                               precision=_HI, preferred_element_type=jnp.float32)

    ftw = ft_ref[...]
    ft_s = _dotC0(Esub, ftw)
    ft_o = _dotC0(Eobj, ftw)
    ftT = jnp.minimum(ft_s, ft_o)
    is_sub = (ft_s <= ft_o).astype(jnp.float32)
    seen = (ftT < BIGF).astype(jnp.float32)
    ft = jnp.transpose(ftT, (1, 0))

    cls_r = lax.broadcasted_iota(jnp.int32, (NCLS, NCLS), 1)
    cls_c = lax.broadcasted_iota(jnp.int32, (NCLS, NCLS), 0)
    cmp = (ftT < ft) | ((ftT == ft) & (cls_c < cls_r))
    rank = jnp.sum(cmp.astype(jnp.int32), axis=0, keepdims=True)
    P = (lax.broadcasted_iota(jnp.int32, (NCLS, NCLS), 0) == rank)
    Pf = P.astype(jnp.float32)

    Cw = c_ref[...]
    C = mm(_dotC0(Eobj, Cw), Esub)
    E = _dotC0(Eobj, e_ref[...])

    x0w = x0_ref[...]
    x0_s = mm(_dotC0(Esub, x0w), Esub)
    x0_o = mm(_dotC0(Eobj, x0w), Eobj)
    x0 = x0_s * is_sub + x0_o * (1.0 - is_sub)
    x0 = x0 * seen + mm(lr_ref[...], Esub) * (1.0 - seen)

    cnt1 = jnp.sum(C, axis=1, keepdims=True) + 1.0

    agg1 = mm(C, x0) + x0 + _dotT(E, wea1_ref[...]) + cnt1 * bea1_ref[...]
    x1 = _dotT(agg1, wl1_ref[...]) + bl1_ref[...]
    agg2 = mm(C, x1) + x1 + _dotT(E, wea2_ref[...]) + cnt1 * bea2_ref[...]
    x2 = _dotT(agg2, wl2_ref[...]) + bl2_ref[...]
    oc = _dotT(x2, wout_ref[...]) + bout_ref[...]
    out_ref[...] = mm(Pf, oc)


@functools.partial(jax.jit, static_argnames=("interpret",))
def _run(scene_feat, W_ea1, b_ea1, W_lin1, b_lin1, W_ea2, b_ea2,
         W_lin2, b_lin2, W_out, b_out, interpret=False):
    f32 = jnp.float32
    sft = scene_feat.T
    Cw, Ew, ftw, x0w = pl.pallas_call(
        _stream_kernel,
        grid=(GRID,),
        in_specs=[pl.BlockSpec((FEAT, LBLK), lambda i: (0, i))],
        out_specs=[
            pl.BlockSpec((FEAT, FEAT), lambda i: (0, 0)),
            pl.BlockSpec((FEAT, EDGE), lambda i: (0, 0)),
            pl.BlockSpec((FEAT, 1), lambda i: (0, 0)),
            pl.BlockSpec((FEAT, FEAT), lambda i: (0, 0)),
        ],
        out_shape=[
            jax.ShapeDtypeStruct((FEAT, FEAT), f32),
            jax.ShapeDtypeStruct((FEAT, EDGE), f32),
            jax.ShapeDtypeStruct((FEAT, 1), f32),
            jax.ShapeDtypeStruct((FEAT, FEAT), f32),
        ],
        interpret=interpret,
    )(sft)

    last_row = lax.slice(scene_feat, (N_REL - 1, 0), (N_REL, FEAT))
    out = pl.pallas_call(
        _finish_kernel,
        out_shape=jax.ShapeDtypeStruct((NCLS, NCLS), f32),
        interpret=interpret,
    )(Cw, Ew, ftw, x0w, last_row,
      W_ea1, b_ea1.reshape(1, -1), W_lin1, b_lin1.reshape(1, -1),
      W_ea2, b_ea2.reshape(1, -1), W_lin2, b_lin2.reshape(1, -1),
      W_out, b_out.reshape(1, -1))
    return out


def kernel(scene_feat, W_ea1, b_ea1, W_lin1, b_lin1, W_ea2, b_ea2,
           W_lin2, b_lin2, W_out, b_out):
    return _run(scene_feat, W_ea1, b_ea1, W_lin1, b_lin1,
                W_ea2, b_ea2, W_lin2, b_lin2, W_out, b_out)


# speedup vs baseline: 4.0729x; 1.7096x over previous
import functools

import jax
import jax.numpy as jnp
from jax import lax
from jax.experimental import pallas as pl

N_REL = 100000
FEAT = 353
NCLS = 151
EDGE = 51
LBLK = 4096
GRID = (N_REL + LBLK - 1) // LBLK
BIGF = float(2 ** 24)

_HI = lax.Precision.HIGHEST


def _dotT(a, b):
    return lax.dot_general(a, b, (((1,), (1,)), ((), ())),
                           precision=_HI, preferred_element_type=jnp.float32)


def _dotL(a, b):
    return lax.dot_general(a, b, (((1,), (1,)), ((), ())),
                           preferred_element_type=jnp.float32)


def _dotC0(a, b):
    return lax.dot_general(a, b, (((0,), (0,)), ((), ())),
                           precision=_HI, preferred_element_type=jnp.float32)


def _stream_kernel(x_ref, c_ref, e_ref, ft_ref, x0_ref):
    pid = pl.program_id(0)

    @pl.when(pid == 0)
    def _init():
        c_ref[...] = jnp.zeros_like(c_ref)
        e_ref[...] = jnp.zeros_like(e_ref)
        r = lax.broadcasted_iota(jnp.int32, (FEAT, 1), 0)
        ft_ref[...] = jnp.where(r >= 51, BIGF, 0.0)
        x0_ref[...] = jnp.zeros_like(x0_ref)

    lane = lax.broadcasted_iota(jnp.int32, (1, LBLK), 1)
    nvalid = jnp.minimum(N_REL - pid * LBLK, LBLK)
    valid = lane < nvalid
    x = x_ref[...]
    rowc = lax.broadcasted_iota(jnp.int32, (FEAT, 1), 0)
    ps = x[51:202, :]
    po = x[202:353, :]
    ps_max = jnp.max(ps, axis=0, keepdims=True)
    po_max = jnp.max(po, axis=0, keepdims=True)
    eq_s = (ps == ps_max) & valid
    eq_o = (po == po_max) & valid
    s_bf = eq_s.astype(jnp.bfloat16)
    d_bf = eq_o.astype(jnp.bfloat16)

    C_blk = _dotL(d_bf, s_bf)
    ea = jnp.where(valid, x[0:EDGE, :], 0.0)
    ea_hi = ea.astype(jnp.bfloat16)
    ea_lo = (ea - ea_hi.astype(jnp.float32)).astype(jnp.bfloat16)
    E_blk = _dotL(d_bf, ea_hi) + _dotL(d_bf, ea_lo)

    c_ref[...] += C_blk
    e_ref[...] += E_blk

    BIGI = 2 ** 24
    rown = lax.broadcasted_iota(jnp.int32, (NCLS, 1), 0)

    total = jnp.sum(C_blk)

    @pl.when(total != nvalid.astype(jnp.float32))
    def _tie_fixup():
        s_row = jnp.min(jnp.where(eq_s, rown, BIGI), axis=0, keepdims=True)
        o_row = jnp.min(jnp.where(eq_o, rown, BIGI), axis=0, keepdims=True)
        s_ex = ((rown == s_row) & valid).astype(jnp.bfloat16)
        d_ex = ((rown == o_row) & valid).astype(jnp.bfloat16)
        C_ex = _dotL(d_ex, s_ex)
        E_ex = _dotL(d_ex, ea_hi) + _dotL(d_ex, ea_lo)
        c_ref[...] += C_ex - C_blk
        e_ref[...] += E_ex - E_blk

    prev = ft_ref[...]

    @pl.when(jnp.max(prev) >= BIGF)
    def _first_occurrence():
        s_row = jnp.min(jnp.where(eq_s, rown, BIGI), axis=0, keepdims=True)
        o_row = jnp.min(jnp.where(eq_o, rown, BIGI), axis=0, keepdims=True)
        s_ex = (rowc == s_row + 51) & valid
        d_ex = (rowc == o_row + 202) & valid
        ex = s_ex | d_ex
        coli = lane + pid * LBLK
        cmin = jnp.min(jnp.where(ex, coli, BIGI), axis=1, keepdims=True)
        par = jnp.where(rowc >= 202, 1, 0)
        cand = jnp.where(cmin < BIGI, (2 * cmin + par).astype(jnp.float32), BIGF)
        newly = cand < prev
        G = (ex & (coli == cmin) & newly)
        g_bf = G.astype(jnp.bfloat16)
        xz = jnp.where(valid, x, 0.0)
        x_hi = xz.astype(jnp.bfloat16)
        x_lo = (xz - x_hi.astype(jnp.float32)).astype(jnp.bfloat16)
        x0_new = _dotL(g_bf, x_hi) + _dotL(g_bf, x_lo)
        m = newly.astype(jnp.float32)
        x0_ref[...] = x0_ref[...] * (1.0 - m) + x0_new * m
        ft_ref[...] = jnp.minimum(prev, cand)


def _finish_kernel(c_ref, e_ref, ft_ref, x0_ref, lr_ref,
                   wea1_ref, bea1_ref, wl1_ref, bl1_ref,
                   wea2_ref, bea2_ref, wl2_ref, bl2_ref,
                   wout_ref, bout_ref, out_ref):
    j353 = lax.broadcasted_iota(jnp.int32, (FEAT, NCLS), 0)
    c353 = lax.broadcasted_iota(jnp.int32, (FEAT, NCLS), 1)
    Esub = (j353 == c353 + 51).astype(jnp.float32)
    Eobj = (j353 == c353 + 202).astype(jnp.float32)

    def mm(a, b):
        return lax.dot_general(a, b, (((1,), (0,)), ((), ())),
                               precision=_HI, preferred_element_type=jnp.float32)

    ftw = ft_ref[...]
    ft_s = _dotC0(Esub, ftw)
    ft_o = _dotC0(Eobj, ftw)
    ftT = jnp.minimum(ft_s, ft_o)
    is_sub = (ft_s <= ft_o).astype(jnp.float32)
    seen = (ftT < BIGF).astype(jnp.float32)
    ft = jnp.transpose(ftT, (1, 0))

    cls_r = lax.broadcasted_iota(jnp.int32, (NCLS, NCLS), 1)
    cls_c = lax.broadcasted_iota(jnp.int32, (NCLS, NCLS), 0)
    cmp = (ftT < ft) | ((ftT == ft) & (cls_c < cls_r))
    rank = jnp.sum(cmp.astype(jnp.int32), axis=0, keepdims=True)
    P = (lax.broadcasted_iota(jnp.int32, (NCLS, NCLS), 0) == rank)
    Pf = P.astype(jnp.float32)

    C = c_ref[...]
    E = e_ref[...]

    x0w = x0_ref[...]
    x0_s = mm(_dotC0(Esub, x0w), Esub)
    x0_o = mm(_dotC0(Eobj, x0w), Eobj)
    x0 = x0_s * is_sub + x0_o * (1.0 - is_sub)
    x0 = x0 * seen + mm(lr_ref[...], Esub) * (1.0 - seen)

    cnt1 = jnp.sum(C, axis=1, keepdims=True) + 1.0

    agg1 = mm(C, x0) + x0 + _dotT(E, wea1_ref[...]) + cnt1 * bea1_ref[...]
    x1 = _dotT(agg1, wl1_ref[...]) + bl1_ref[...]
    agg2 = mm(C, x1) + x1 + _dotT(E, wea2_ref[...]) + cnt1 * bea2_ref[...]
    x2 = _dotT(agg2, wl2_ref[...]) + bl2_ref[...]
    oc = _dotT(x2, wout_ref[...]) + bout_ref[...]
    out_ref[...] = mm(Pf, oc)


@functools.partial(jax.jit, static_argnames=("interpret",))
def _run(scene_feat, W_ea1, b_ea1, W_lin1, b_lin1, W_ea2, b_ea2,
         W_lin2, b_lin2, W_out, b_out, interpret=False):
    f32 = jnp.float32
    sft = scene_feat.T
    Cw, Ew, ftw, x0w = pl.pallas_call(
        _stream_kernel,
        grid=(GRID,),
        in_specs=[pl.BlockSpec((FEAT, LBLK), lambda i: (0, i))],
        out_specs=[
            pl.BlockSpec((NCLS, NCLS), lambda i: (0, 0)),
            pl.BlockSpec((NCLS, EDGE), lambda i: (0, 0)),
            pl.BlockSpec((FEAT, 1), lambda i: (0, 0)),
            pl.BlockSpec((FEAT, FEAT), lambda i: (0, 0)),
        ],
        out_shape=[
            jax.ShapeDtypeStruct((NCLS, NCLS), f32),
            jax.ShapeDtypeStruct((NCLS, EDGE), f32),
            jax.ShapeDtypeStruct((FEAT, 1), f32),
            jax.ShapeDtypeStruct((FEAT, FEAT), f32),
        ],
        interpret=interpret,
    )(sft)

    last_row = lax.slice(scene_feat, (N_REL - 1, 0), (N_REL, FEAT))
    out = pl.pallas_call(
        _finish_kernel,
        out_shape=jax.ShapeDtypeStruct((NCLS, NCLS), f32),
        interpret=interpret,
    )(Cw, Ew, ftw, x0w, last_row,
      W_ea1, b_ea1.reshape(1, -1), W_lin1, b_lin1.reshape(1, -1),
      W_ea2, b_ea2.reshape(1, -1), W_lin2, b_lin2.reshape(1, -1),
      W_out, b_out.reshape(1, -1))
    return out


def kernel(scene_feat, W_ea1, b_ea1, W_lin1, b_lin1, W_ea2, b_ea2,
           W_lin2, b_lin2, W_out, b_out):
    return _run(scene_feat, W_ea1, b_ea1, W_lin1, b_lin1,
                W_ea2, b_ea2, W_lin2, b_lin2, W_out, b_out)
